# Initial kernel scaffold; baseline (speedup 1.0000x reference)
import jax
import jax.numpy as jnp
from jax.experimental import pallas as pl


def kernel(x, w_gate):
    raise NotImplementedError("write your pallas kernel here")


# speedup vs baseline: 3.5342x; 3.5342x over previous
import jax
import jax.numpy as jnp
from jax import lax
from jax.experimental import pallas as pl
from jax.experimental.pallas import tpu as pltpu

D_MODEL = 4096
N_EXP = 64
TOPK = 8
BT = 256


def _fused_body(x_ref, w_ref, gates_ref, idx_ref, aux_ref, gacc, pacc):
    i = pl.program_id(0)
    nb = pl.num_programs(0)

    logits = lax.dot_general(
        x_ref[...], w_ref[...], (((1,), (1,)), ((), ())),
        preferred_element_type=jnp.float32)

    rowmax = jnp.max(logits, axis=1, keepdims=True)
    p = jnp.exp(logits - rowmax)
    probs = p / jnp.sum(p, axis=1, keepdims=True)

    iota = lax.broadcasted_iota(jnp.int32, logits.shape, 1)
    l = logits
    sel = jnp.zeros(logits.shape, jnp.bool_)
    idx_cols = []
    for _ in range(TOPK):
        m = jnp.max(l, axis=1, keepdims=True)
        idx = jnp.min(jnp.where(l == m, iota, N_EXP), axis=1, keepdims=True)
        first = iota == idx
        sel = sel | first
        idx_cols.append(idx)
        l = jnp.where(first, -jnp.inf, l)

    psel = jnp.where(sel, p, 0.0)
    gates = psel / jnp.sum(psel, axis=1, keepdims=True)

    gates_ref[...] = gates
    idx_ref[...] = jnp.concatenate(idx_cols, axis=1).astype(jnp.int32)

    gpart = jnp.sum(gates, axis=0, keepdims=True)
    ppart = jnp.sum(probs, axis=0, keepdims=True)

    @pl.when(i == 0)
    def _init():
        gacc[...] = gpart
        pacc[...] = ppart

    @pl.when(i > 0)
    def _acc():
        gacc[...] += gpart
        pacc[...] += ppart

    @pl.when(i == nb - 1)
    def _fin():
        t = jnp.float32(nb * BT)
        aux_ref[0, 0] = jnp.sum(gacc[...] * pacc[...]) * (N_EXP / (t * t))


def kernel(x, w_gate):
    t_rows = x.shape[0]
    nb = t_rows // BT
    gates, idx, aux = pl.pallas_call(
        _fused_body,
        grid=(nb,),
        in_specs=[
            pl.BlockSpec((BT, D_MODEL), lambda i: (i, 0)),
            pl.BlockSpec((N_EXP, D_MODEL), lambda i: (0, 0)),
        ],
        out_specs=[
            pl.BlockSpec((BT, N_EXP), lambda i: (i, 0)),
            pl.BlockSpec((BT, TOPK), lambda i: (i, 0)),
            pl.BlockSpec(memory_space=pltpu.SMEM),
        ],
        out_shape=[
            jax.ShapeDtypeStruct((t_rows, N_EXP), jnp.float32),
            jax.ShapeDtypeStruct((t_rows, TOPK), jnp.int32),
            jax.ShapeDtypeStruct((1, 1), jnp.float32),
        ],
        scratch_shapes=[
            pltpu.VMEM((1, N_EXP), jnp.float32),
            pltpu.VMEM((1, N_EXP), jnp.float32),
        ],
    )(x, w_gate)
    return (gates, idx, aux[0, 0])


# speedup vs baseline: 4.8511x; 1.3726x over previous
import jax
import jax.numpy as jnp
from jax import lax
from jax.experimental import pallas as pl
from jax.experimental.pallas import tpu as pltpu

D_MODEL = 4096
N_EXP = 64
TOPK = 8
BT = 256


def _fused_body(x_ref, w_ref, gates_ref, idx_ref, aux_ref, gacc, pacc):
    i = pl.program_id(0)
    nb = pl.num_programs(0)

    logits = lax.dot_general(
        x_ref[...], w_ref[...], (((1,), (1,)), ((), ())),
        preferred_element_type=jnp.float32)

    rowmax = jnp.max(logits, axis=1, keepdims=True)
    p = jnp.exp(logits - rowmax)
    probs = p / jnp.sum(p, axis=1, keepdims=True)

    iota = lax.broadcasted_iota(jnp.int32, logits.shape, 1)
    key = lax.bitcast_convert_type(
        (lax.bitcast_convert_type(logits, jnp.int32) & jnp.int32(~63))
        | (jnp.int32(63) - iota), jnp.float32)

    lk = key
    key_cols = []
    for j in range(TOPK):
        m = jnp.max(lk, axis=1, keepdims=True)
        key_cols.append(m)
        if j < TOPK - 1:
            lk = jnp.where(lk == m, -jnp.inf, lk)

    thresh = key_cols[-1]
    psel = jnp.where(key >= thresh, p, 0.0)
    gates = psel / jnp.sum(psel, axis=1, keepdims=True)

    gates_ref[...] = gates
    keys8 = jnp.concatenate(key_cols, axis=1)
    idx_ref[...] = jnp.int32(63) - (
        lax.bitcast_convert_type(keys8, jnp.int32) & jnp.int32(63))

    gpart = jnp.sum(gates, axis=0, keepdims=True)
    ppart = jnp.sum(probs, axis=0, keepdims=True)

    @pl.when(i == 0)
    def _init():
        gacc[...] = gpart
        pacc[...] = ppart

    @pl.when(i > 0)
    def _acc():
        gacc[...] += gpart
        pacc[...] += ppart

    @pl.when(i == nb - 1)
    def _fin():
        t = jnp.float32(nb * BT)
        aux_ref[0, 0] = jnp.sum(gacc[...] * pacc[...]) * (N_EXP / (t * t))


def kernel(x, w_gate):
    t_rows = x.shape[0]
    nb = t_rows // BT
    gates, idx, aux = pl.pallas_call(
        _fused_body,
        grid=(nb,),
        in_specs=[
            pl.BlockSpec((BT, D_MODEL), lambda i: (i, 0)),
            pl.BlockSpec((N_EXP, D_MODEL), lambda i: (0, 0)),
        ],
        out_specs=[
            pl.BlockSpec((BT, N_EXP), lambda i: (i, 0)),
            pl.BlockSpec((BT, TOPK), lambda i: (i, 0)),
            pl.BlockSpec(memory_space=pltpu.SMEM),
        ],
        out_shape=[
            jax.ShapeDtypeStruct((t_rows, N_EXP), jnp.float32),
            jax.ShapeDtypeStruct((t_rows, TOPK), jnp.int32),
            jax.ShapeDtypeStruct((1, 1), jnp.float32),
        ],
        scratch_shapes=[
            pltpu.VMEM((1, N_EXP), jnp.float32),
            pltpu.VMEM((1, N_EXP), jnp.float32),
        ],
    )(x, w_gate)
    return (gates, idx, aux[0, 0])
